# baseline (speedup 1.0000x reference)
import functools

import jax
import jax.numpy as jnp
from jax import lax
from jax.experimental import pallas as pl
from jax.experimental.pallas import tpu as pltpu
from jax.experimental.pallas import tpu_sc as plsc

_N, _D, _K, _S = 10000, 256, 8, 1250
_NE = 125
_TH = 12
_CP = 128
_B = _K * _CP
_NC, _NS = 2, 16
_NW = _NC * _NS
_BW = _B // _NW
_BLK = 2000
_G = _N // _BLK
_MP = 1024
_MBUF = 10240
_EPS_PD = 1e-6
_EPS_COS = 1e-8
_HI = lax.Precision.HIGHEST


def _sc_gather_mask(node_fea, allidx):
    mesh = plsc.VectorSubcoreMesh(
        core_axis_name="c", subcore_axis_name="s",
        num_cores=_NC, num_subcores=_NS)

    @functools.partial(
        pl.kernel,
        out_type=(jax.ShapeDtypeStruct((_B, _D), jnp.float32),
                  jax.ShapeDtypeStruct((_N,), jnp.float32)),
        mesh=mesh,
        scratch_types=[
            pltpu.VMEM((_BW,), jnp.int32),
            pltpu.VMEM((_BW, _D), jnp.float32),
            pltpu.SemaphoreType.DMA,
            pltpu.VMEM((_MBUF,), jnp.float32),
            pltpu.VMEM((_MP,), jnp.int32),
        ],
        compiler_params=pltpu.CompilerParams(needs_layout_passes=False),
    )
    def sc_k(table_hbm, allidx_hbm, fea_out, mask_out,
             idx_v, rows_v, sem, mbuf, midx_v):
        wid = lax.axis_index("s") * _NC + lax.axis_index("c")
        base = wid * _BW
        pltpu.sync_copy(allidx_hbm.at[pl.ds(base, _BW)], idx_v)
        pltpu.async_copy(table_hbm.at[idx_v], rows_v, sem).wait()
        pltpu.sync_copy(rows_v, fea_out.at[pl.ds(base, _BW)])

        @pl.when(wid == 1)
        def _mask_work():
            zeros16 = jnp.zeros((16,), jnp.float32)
            ones16 = jnp.ones((16,), jnp.float32)

            def zbody(i, carry):
                mbuf[pl.ds(i * 16, 16)] = zeros16
                return carry

            lax.fori_loop(0, _MBUF // 16, zbody, 0)
            pltpu.sync_copy(allidx_hbm.at[pl.ds(_B, _MP)], midx_v)

            def sbody(j, carry):
                idx16 = midx_v[pl.ds(j * 16, 16)]
                plsc.store_scatter(mbuf, [idx16], ones16)
                return carry

            lax.fori_loop(0, _MP // 16, sbody, 0)
            pltpu.sync_copy(mbuf.at[pl.ds(0, _N)], mask_out)

    return sc_k(node_fea, allidx)


def _tc_a_body(x_ref, lab_ref, cent_ref, base_ref, d_ref):
    i = pl.program_id(0)

    @pl.when(i == 0)
    def _init():
        base_ref[...] = jnp.zeros((1, 1), jnp.float32)

    x = x_ref[...]
    lab_row = lab_ref[0].astype(jnp.float32)
    lab_col = lax.transpose(lab_row, (1, 0))
    oh = (lab_col == lax.broadcasted_iota(jnp.int32, (_BLK, _K), 1
                                          ).astype(jnp.float32))
    csel = lax.dot_general(oh.astype(jnp.float32), cent_ref[...],
                           (((1,), (0,)), ((), ())))
    df = x - csel + _EPS_PD
    ssq = jnp.sum(df * df, axis=1, keepdims=True)
    dd = jnp.sqrt(ssq)
    d_ref[...] = lax.transpose(dd, (1, 0)).reshape(1, 1, _BLK)
    part = jnp.sum(dd, axis=0, keepdims=True)
    base_ref[...] = base_ref[...] + part


def _tc_a(x, lab3, cent):
    return pl.pallas_call(
        _tc_a_body,
        grid=(_G,),
        in_specs=[
            pl.BlockSpec((_BLK, _D), lambda i: (i, 0)),
            pl.BlockSpec((1, 1, _BLK), lambda i: (i, 0, 0)),
            pl.BlockSpec((_K, _D), lambda i: (0, 0)),
        ],
        out_specs=[
            pl.BlockSpec((1, 1), lambda i: (0, 0)),
            pl.BlockSpec((1, 1, _BLK), lambda i: (i, 0, 0)),
        ],
        out_shape=[
            jax.ShapeDtypeStruct((1, 1), jnp.float32),
            jax.ShapeDtypeStruct((_G, 1, _BLK), jnp.float32),
        ],
    )(x, lab3, cent)


def _tc_b_body(base_ref, d_ref, mask_ref, fea_ref, cent_ref, scale_ref,
               out_ref, means_s, cnts_s):
    c = pl.program_id(0)
    cent = cent_ref[...]
    ones_row = jnp.ones((1, _D), jnp.float32)
    eye = (lax.broadcasted_iota(jnp.int32, (_CP, _CP), 0)
           == lax.broadcasted_iota(jnp.int32, (_CP, _CP), 1)
           ).astype(jnp.float32)
    valid_row = lax.broadcasted_iota(jnp.int32, (1, _CP), 1) < _NE
    valid_col = lax.broadcasted_iota(jnp.int32, (_CP, 1), 0) < _NE
    zrow = jnp.zeros((1, _CP), jnp.float32)

    @pl.when(c < _K)
    def _cluster():
        fc = fea_ref[...]
        atc = lax.dot_general(cent, fc,
                              (((1,), (1,)), ((), ())))
        n2 = lax.dot_general(ones_row, fc * fc,
                             (((1,), (1,)), ((), ())))
        invn = 1.0 / jnp.maximum(jnp.sqrt(n2), _EPS_COS)
        ohc = (lax.broadcasted_iota(jnp.int32, (1, _K), 1) == c
               ).astype(jnp.float32)
        crow = lax.dot_general(ohc, atc, (((1,), (0,)), ((), ())),
                               precision=_HI)
        srow = (crow - atc) * invn
        keeps = []
        for j in range(_K):
            srow_j = lax.slice(srow, (j, 0), (j + 1, _CP))
            scol_j = lax.dot_general(eye, srow_j,
                                     (((1,), (1,)), ((), ())))
            less = (scol_j < srow_j) & valid_col
            cntr = jnp.sum(less.astype(jnp.float32), axis=0,
                           keepdims=True)
            kj = (cntr >= float(_TH + 1)) & valid_row
            keeps.append(kj.astype(jnp.float32))
        keep_c = jnp.concatenate(keeps, axis=0)
        own = (lax.broadcasted_iota(jnp.int32, (_K, 1), 0) != c)
        keep_c = keep_c * own.astype(jnp.float32)
        ck = jnp.sum(keep_c, axis=1, keepdims=True)
        ms = lax.dot_general(keep_c, fc,
                             (((1,), (0,)), ((), ())))
        mean_c = ms / jnp.maximum(ck, 1.0)
        means_s[pl.ds(c * _K, _K), :] = mean_c
        cnts_s[pl.ds(c * _K, _K), :] = ck

    @pl.when(c == _K)
    def _final():
        msum = jnp.zeros((1, 1), jnp.float32)
        for b in range(_G):
            prod = d_ref[b] * mask_ref[b]
            msum = msum + jnp.sum(prod, axis=1, keepdims=True)
        total = base_ref[...] + scale_ref[0, 0] * msum
        l2 = jnp.zeros((1, 1), jnp.float32)
        for a in range(_K):
            for b in range(a + 1, _K):
                mi = means_s[pl.ds(a * _K + b, 1), :]
                mj = means_s[pl.ds(b * _K + a, 1), :]
                dfp = mi - mj + _EPS_PD
                dist = jnp.sqrt(jnp.sum(dfp * dfp, axis=1, keepdims=True))
                ca = cnts_s[pl.ds(a * _K + b, 1), :]
                cb = cnts_s[pl.ds(b * _K + a, 1), :]
                ok = (ca > 0.0) & (cb > 0.0)
                l2 = l2 + jnp.where(ok, dist, 0.0)
        out_ref[...] = total - l2


def _tc_b(base, d3, mask3, fea, cent, scale):
    return pl.pallas_call(
        _tc_b_body,
        grid=(_K + 1,),
        in_specs=[
            pl.BlockSpec((1, 1), lambda c: (0, 0)),
            pl.BlockSpec((_G, 1, _BLK), lambda c: (0, 0, 0)),
            pl.BlockSpec((_G, 1, _BLK), lambda c: (0, 0, 0)),
            pl.BlockSpec((_CP, _D), lambda c: (jnp.minimum(c, _K - 1), 0)),
            pl.BlockSpec((_K, _D), lambda c: (0, 0)),
            pl.BlockSpec(memory_space=pltpu.SMEM),
        ],
        out_specs=pl.BlockSpec((1, 1), lambda c: (0, 0)),
        out_shape=jax.ShapeDtypeStruct((1, 1), jnp.float32),
        scratch_shapes=[
            pltpu.VMEM((_K * _K, _D), jnp.float32),
            pltpu.VMEM((_K * _K, 1), jnp.float32),
        ],
    )(base, d3, mask3, fea, cent, scale)


def kernel(node_fea, clu_label, center_fea, mask_nodes, mask_weight,
           sort_idx_rst):
    node_fea = node_fea.astype(jnp.float32)
    center_fea = center_fea.astype(jnp.float32)
    edge_idx = sort_idx_rst[:, _S - _NE:].astype(jnp.int32)
    gidx = jnp.pad(edge_idx, ((0, 0), (0, _CP - _NE))).reshape(_B)
    m = mask_nodes.shape[0]
    allidx = jnp.concatenate([
        gidx,
        mask_nodes.astype(jnp.int32),
        jnp.full((_MP - m,), _MBUF - 8, jnp.int32),
    ])
    fea_pad, mask = _sc_gather_mask(node_fea, allidx)
    scale = (jnp.asarray(mask_weight, jnp.float32) + 1.0).reshape(1, 1)
    lab3 = clu_label.astype(jnp.int32).reshape(_G, 1, _BLK)
    base, d3 = _tc_a(node_fea, lab3, center_fea)
    out = _tc_b(base, d3, mask.reshape(_G, 1, _BLK), fea_pad,
                center_fea, scale)
    return out.reshape(1)


# speedup vs baseline: 1.0633x; 1.0633x over previous
import functools

import jax
import jax.numpy as jnp
from jax import lax
from jax.experimental import pallas as pl
from jax.experimental.pallas import tpu as pltpu
from jax.experimental.pallas import tpu_sc as plsc

_N, _D, _K, _S = 10000, 256, 8, 1250
_NE = 125
_TH = 12
_CP = 128
_B = _K * _CP
_NC, _NS = 2, 16
_NW = _NC * _NS
_BW = _B // _NW
_BLK = 2000
_G = _N // _BLK
_MP = 1024
_MBUF = 10240
_EPS_PD = 1e-6
_EPS_COS = 1e-8
_HI = lax.Precision.HIGHEST


def _sc_gather_mask(node_fea, allidx):
    mesh = plsc.VectorSubcoreMesh(
        core_axis_name="c", subcore_axis_name="s",
        num_cores=_NC, num_subcores=_NS)

    @functools.partial(
        pl.kernel,
        out_type=(jax.ShapeDtypeStruct((_B, _D), jnp.float32),
                  jax.ShapeDtypeStruct((_N,), jnp.float32)),
        mesh=mesh,
        scratch_types=[
            pltpu.VMEM((_BW,), jnp.int32),
            pltpu.VMEM((_BW, _D), jnp.float32),
            pltpu.SemaphoreType.DMA,
            pltpu.VMEM((_MBUF,), jnp.float32),
            pltpu.VMEM((_MP,), jnp.int32),
        ],
        compiler_params=pltpu.CompilerParams(needs_layout_passes=False),
    )
    def sc_k(table_hbm, allidx_hbm, fea_out, mask_out,
             idx_v, rows_v, sem, mbuf, midx_v):
        wid = lax.axis_index("s") * _NC + lax.axis_index("c")
        base = wid * _BW
        pltpu.sync_copy(allidx_hbm.at[pl.ds(base, _BW)], idx_v)
        pltpu.async_copy(table_hbm.at[idx_v], rows_v, sem).wait()
        pltpu.sync_copy(rows_v, fea_out.at[pl.ds(base, _BW)])

        @pl.when(wid == 1)
        def _mask_work():
            zeros16 = jnp.zeros((16,), jnp.float32)
            ones16 = jnp.ones((16,), jnp.float32)

            def zbody(i, carry):
                mbuf[pl.ds(i * 16, 16)] = zeros16
                return carry

            lax.fori_loop(0, _MBUF // 16, zbody, 0)
            pltpu.sync_copy(allidx_hbm.at[pl.ds(_B, _MP)], midx_v)

            def sbody(j, carry):
                idx16 = midx_v[pl.ds(j * 16, 16)]
                plsc.store_scatter(mbuf, [idx16], ones16)
                return carry

            lax.fori_loop(0, _MP // 16, sbody, 0)
            pltpu.sync_copy(mbuf.at[pl.ds(0, _N)], mask_out)

    return sc_k(node_fea, allidx)


def _tc_a_body(x_ref, lab_ref, cent_ref, base_ref, d_ref):
    i = pl.program_id(0)

    @pl.when(i == 0)
    def _init():
        base_ref[...] = jnp.zeros((1, 1), jnp.float32)

    x = x_ref[...]
    lab_row = lab_ref[0].astype(jnp.float32)
    lab_col = lax.transpose(lab_row, (1, 0))
    oh = (lab_col == lax.broadcasted_iota(jnp.int32, (_BLK, _K), 1
                                          ).astype(jnp.float32))
    csel = lax.dot_general(oh.astype(jnp.float32), cent_ref[...],
                           (((1,), (0,)), ((), ())))
    df = x - csel + _EPS_PD
    ssq = jnp.sum(df * df, axis=1, keepdims=True)
    dd = jnp.sqrt(ssq)
    d_ref[...] = lax.transpose(dd, (1, 0)).reshape(1, 1, _BLK)
    part = jnp.sum(dd, axis=0, keepdims=True)
    base_ref[...] = base_ref[...] + part


def _tc_a(x, lab3, cent):
    return pl.pallas_call(
        _tc_a_body,
        grid=(_G,),
        in_specs=[
            pl.BlockSpec((_BLK, _D), lambda i: (i, 0)),
            pl.BlockSpec((1, 1, _BLK), lambda i: (i, 0, 0)),
            pl.BlockSpec((_K, _D), lambda i: (0, 0)),
        ],
        out_specs=[
            pl.BlockSpec((1, 1), lambda i: (0, 0)),
            pl.BlockSpec((1, 1, _BLK), lambda i: (i, 0, 0)),
        ],
        out_shape=[
            jax.ShapeDtypeStruct((1, 1), jnp.float32),
            jax.ShapeDtypeStruct((_G, 1, _BLK), jnp.float32),
        ],
    )(x, lab3, cent)


def _tc_b_body(base_ref, d_ref, mask_ref, fea_ref, cent_ref, scale_ref,
               out_ref):
    msum = jnp.zeros((1, 1), jnp.float32)
    for b in range(_G):
        prod = d_ref[b] * mask_ref[b]
        msum = msum + jnp.sum(prod, axis=1, keepdims=True)
    total = base_ref[...] + scale_ref[0, 0] * msum

    fea = fea_ref[...]
    cent = cent_ref[...]
    at = lax.dot_general(cent, fea,
                         (((1,), (1,)), ((), ())))
    ones_row = jnp.ones((1, _D), jnp.float32)
    eye = (lax.broadcasted_iota(jnp.int32, (_CP, _CP), 0)
           == lax.broadcasted_iota(jnp.int32, (_CP, _CP), 1)
           ).astype(jnp.float32)
    valid_row = lax.broadcasted_iota(jnp.int32, (1, _CP), 1) < _NE
    valid_col = lax.broadcasted_iota(jnp.int32, (_CP, 1), 0) < _NE
    zrow = jnp.zeros((1, _CP), jnp.float32)
    means = []
    cnts = []
    for c in range(_K):
        fc = lax.slice(fea, (c * _CP, 0), ((c + 1) * _CP, _D))
        atc = lax.slice(at, (0, c * _CP), (_K, (c + 1) * _CP))
        n2 = lax.dot_general(ones_row, fc * fc,
                             (((1,), (1,)), ((), ())))
        invn = 1.0 / jnp.maximum(jnp.sqrt(n2), _EPS_COS)
        srow = (lax.slice(atc, (c, 0), (c + 1, _CP)) - atc) * invn
        keeps = []
        for j in range(_K):
            if j == c:
                keeps.append(zrow)
                continue
            srow_j = lax.slice(srow, (j, 0), (j + 1, _CP))
            scol_j = lax.dot_general(eye, srow_j,
                                     (((1,), (1,)), ((), ())))
            less = (scol_j < srow_j) & valid_col
            cntr = jnp.sum(less.astype(jnp.float32), axis=0,
                           keepdims=True)
            kj = (cntr >= float(_TH + 1)) & valid_row
            keeps.append(kj.astype(jnp.float32))
        keep_c = jnp.concatenate(keeps, axis=0)
        ck = jnp.sum(keep_c, axis=1, keepdims=True)
        ms = lax.dot_general(keep_c, fc,
                             (((1,), (0,)), ((), ())))
        means.append(ms / jnp.maximum(ck, 1.0))
        cnts.append(ck)
    l2 = jnp.zeros((1, 1), jnp.float32)
    for a in range(_K):
        for b in range(a + 1, _K):
            mi = lax.slice(means[a], (b, 0), (b + 1, _D))
            mj = lax.slice(means[b], (a, 0), (a + 1, _D))
            dfp = mi - mj + _EPS_PD
            dist = jnp.sqrt(jnp.sum(dfp * dfp, axis=1, keepdims=True))
            ca = lax.slice(cnts[a], (b, 0), (b + 1, 1))
            cb = lax.slice(cnts[b], (a, 0), (a + 1, 1))
            ok = (ca > 0.0) & (cb > 0.0)
            l2 = l2 + jnp.where(ok, dist, 0.0)
    out_ref[...] = total - l2


def _tc_b(base, d3, mask3, fea, cent, scale):
    return pl.pallas_call(
        _tc_b_body,
        in_specs=[
            pl.BlockSpec((1, 1), lambda: (0, 0)),
            pl.BlockSpec((_G, 1, _BLK), lambda: (0, 0, 0)),
            pl.BlockSpec((_G, 1, _BLK), lambda: (0, 0, 0)),
            pl.BlockSpec((_B, _D), lambda: (0, 0)),
            pl.BlockSpec((_K, _D), lambda: (0, 0)),
            pl.BlockSpec(memory_space=pltpu.SMEM),
        ],
        out_specs=pl.BlockSpec((1, 1), lambda: (0, 0)),
        out_shape=jax.ShapeDtypeStruct((1, 1), jnp.float32),
    )(base, d3, mask3, fea, cent, scale)


def kernel(node_fea, clu_label, center_fea, mask_nodes, mask_weight,
           sort_idx_rst):
    node_fea = node_fea.astype(jnp.float32)
    center_fea = center_fea.astype(jnp.float32)
    edge_idx = sort_idx_rst[:, _S - _NE:].astype(jnp.int32)
    gidx = jnp.pad(edge_idx, ((0, 0), (0, _CP - _NE))).reshape(_B)
    m = mask_nodes.shape[0]
    allidx = jnp.concatenate([
        gidx,
        mask_nodes.astype(jnp.int32),
        jnp.full((_MP - m,), _MBUF - 8, jnp.int32),
    ])
    fea_pad, mask = _sc_gather_mask(node_fea, allidx)
    scale = (jnp.asarray(mask_weight, jnp.float32) + 1.0).reshape(1, 1)
    lab3 = clu_label.astype(jnp.int32).reshape(_G, 1, _BLK)
    base, d3 = _tc_a(node_fea, lab3, center_fea)
    out = _tc_b(base, d3, mask.reshape(_G, 1, _BLK), fea_pad,
                center_fea, scale)
    return out.reshape(1)


# speedup vs baseline: 1.1297x; 1.0624x over previous
import functools

import jax
import jax.numpy as jnp
from jax import lax
from jax.experimental import pallas as pl
from jax.experimental.pallas import tpu as pltpu
from jax.experimental.pallas import tpu_sc as plsc

_N, _D, _K, _S = 10000, 256, 8, 1250
_NE = 125
_TH = 12
_CP = 128
_B = _K * _CP
_NC, _NS = 2, 16
_NW = _NC * _NS
_BW = _B // _NW
_BLK = 2048
_G = 5
_NP = _G * _BLK
_MP = 1024
_MBUF = 10240
_EPS_PD = 1e-6
_EPS_COS = 1e-8
_HI = lax.Precision.HIGHEST


def _sc_gather_mask(node_fea, allidx):
    mesh = plsc.VectorSubcoreMesh(
        core_axis_name="c", subcore_axis_name="s",
        num_cores=_NC, num_subcores=_NS)

    @functools.partial(
        pl.kernel,
        out_type=(jax.ShapeDtypeStruct((_B, _D), jnp.float32),
                  jax.ShapeDtypeStruct((_NP,), jnp.float32)),
        mesh=mesh,
        scratch_types=[
            pltpu.VMEM((_BW,), jnp.int32),
            pltpu.VMEM((_BW, _D), jnp.float32),
            pltpu.SemaphoreType.DMA,
            pltpu.VMEM((_MBUF,), jnp.float32),
            pltpu.VMEM((_MP,), jnp.int32),
        ],
        compiler_params=pltpu.CompilerParams(needs_layout_passes=False),
    )
    def sc_k(table_hbm, allidx_hbm, fea_out, mask_out,
             idx_v, rows_v, sem, mbuf, midx_v):
        wid = lax.axis_index("s") * _NC + lax.axis_index("c")
        base = wid * _BW
        pltpu.sync_copy(allidx_hbm.at[pl.ds(base, _BW)], idx_v)
        pltpu.async_copy(table_hbm.at[idx_v], rows_v, sem).wait()
        pltpu.sync_copy(rows_v, fea_out.at[pl.ds(base, _BW)])

        @pl.when(wid == 1)
        def _mask_work():
            zeros16 = jnp.zeros((16,), jnp.float32)
            ones16 = jnp.ones((16,), jnp.float32)

            def zbody(i, carry):
                mbuf[pl.ds(i * 16, 16)] = zeros16
                return carry

            lax.fori_loop(0, _MBUF // 16, zbody, 0)
            pltpu.sync_copy(allidx_hbm.at[pl.ds(_B, _MP)], midx_v)

            def sbody(j, carry):
                idx16 = midx_v[pl.ds(j * 16, 16)]
                plsc.store_scatter(mbuf, [idx16], ones16)
                return carry

            lax.fori_loop(0, _MP // 16, sbody, 0)
            pltpu.sync_copy(mbuf, mask_out)

    return sc_k(node_fea, allidx)


def _tc_a_body(x_ref, lab_ref, cent_ref, base_ref, d_ref):
    i = pl.program_id(0)

    @pl.when(i == 0)
    def _init():
        base_ref[...] = jnp.zeros((1, 1), jnp.float32)

    x = x_ref[...]
    lab_col = lab_ref[...].astype(jnp.float32).reshape(_BLK, 1)
    oh = (lab_col == lax.broadcasted_iota(jnp.int32, (_BLK, _K), 1
                                          ).astype(jnp.float32))
    csel = lax.dot_general(oh.astype(jnp.float32), cent_ref[...],
                           (((1,), (0,)), ((), ())))
    df = x - csel + _EPS_PD
    ssq = jnp.sum(df * df, axis=1, keepdims=True)
    valid = (i * _BLK
             + lax.broadcasted_iota(jnp.int32, (_BLK, 1), 0)) < _N
    dd = jnp.where(valid, jnp.sqrt(ssq), 0.0)
    d_ref[...] = dd.reshape(_BLK)
    part = jnp.sum(dd, axis=0, keepdims=True)
    base_ref[...] = base_ref[...] + part


def _tc_a(x, allidx, cent):
    return pl.pallas_call(
        _tc_a_body,
        grid=(_G,),
        in_specs=[
            pl.BlockSpec((_BLK, _D), lambda i: (i, 0)),
            pl.BlockSpec((_BLK,), lambda i: (i + 1,)),
            pl.BlockSpec((_K, _D), lambda i: (0, 0)),
        ],
        out_specs=[
            pl.BlockSpec((1, 1), lambda i: (0, 0)),
            pl.BlockSpec((_BLK,), lambda i: (i,)),
        ],
        out_shape=[
            jax.ShapeDtypeStruct((1, 1), jnp.float32),
            jax.ShapeDtypeStruct((_NP,), jnp.float32),
        ],
    )(x, allidx, cent)


def _tc_b_body(base_ref, d_ref, mask_ref, fea_ref, cent_ref, scale_ref,
               out_ref):
    prod = (d_ref[...] * mask_ref[...]).reshape(_NP // 128, 128)
    msum = jnp.sum(jnp.sum(prod, axis=1, keepdims=True),
                   axis=0, keepdims=True)
    total = base_ref[...] + scale_ref[0, 0] * msum

    fea = fea_ref[...]
    cent = cent_ref[...]
    at = lax.dot_general(cent, fea,
                         (((1,), (1,)), ((), ())))
    ones_row = jnp.ones((1, _D), jnp.float32)
    eye = (lax.broadcasted_iota(jnp.int32, (_CP, _CP), 0)
           == lax.broadcasted_iota(jnp.int32, (_CP, _CP), 1)
           ).astype(jnp.float32)
    valid_row = lax.broadcasted_iota(jnp.int32, (1, _CP), 1) < _NE
    valid_col = lax.broadcasted_iota(jnp.int32, (_CP, 1), 0) < _NE
    zrow = jnp.zeros((1, _CP), jnp.float32)
    means = []
    cnts = []
    for c in range(_K):
        fc = lax.slice(fea, (c * _CP, 0), ((c + 1) * _CP, _D))
        atc = lax.slice(at, (0, c * _CP), (_K, (c + 1) * _CP))
        n2 = lax.dot_general(ones_row, fc * fc,
                             (((1,), (1,)), ((), ())))
        invn = 1.0 / jnp.maximum(jnp.sqrt(n2), _EPS_COS)
        srow = (lax.slice(atc, (c, 0), (c + 1, _CP)) - atc) * invn
        keeps = []
        for j in range(_K):
            if j == c:
                keeps.append(zrow)
                continue
            srow_j = lax.slice(srow, (j, 0), (j + 1, _CP))
            scol_j = lax.dot_general(eye, srow_j,
                                     (((1,), (1,)), ((), ())))
            less = (scol_j < srow_j) & valid_col
            cntr = jnp.sum(less.astype(jnp.float32), axis=0,
                           keepdims=True)
            kj = (cntr >= float(_TH + 1)) & valid_row
            keeps.append(kj.astype(jnp.float32))
        keep_c = jnp.concatenate(keeps, axis=0)
        ck = jnp.sum(keep_c, axis=1, keepdims=True)
        ms = lax.dot_general(keep_c, fc,
                             (((1,), (0,)), ((), ())))
        means.append(ms / jnp.maximum(ck, 1.0))
        cnts.append(ck)
    l2 = jnp.zeros((1, 1), jnp.float32)
    for a in range(_K):
        for b in range(a + 1, _K):
            mi = lax.slice(means[a], (b, 0), (b + 1, _D))
            mj = lax.slice(means[b], (a, 0), (a + 1, _D))
            dfp = mi - mj + _EPS_PD
            dist = jnp.sqrt(jnp.sum(dfp * dfp, axis=1, keepdims=True))
            ca = lax.slice(cnts[a], (b, 0), (b + 1, 1))
            cb = lax.slice(cnts[b], (a, 0), (a + 1, 1))
            ok = (ca > 0.0) & (cb > 0.0)
            l2 = l2 + jnp.where(ok, dist, 0.0)
    out_ref[...] = total - l2


def _tc_b(base, d3, mask3, fea, cent, scale):
    return pl.pallas_call(
        _tc_b_body,
        in_specs=[
            pl.BlockSpec((1, 1), lambda: (0, 0)),
            pl.BlockSpec((_NP,), lambda: (0,)),
            pl.BlockSpec((_NP,), lambda: (0,)),
            pl.BlockSpec((_B, _D), lambda: (0, 0)),
            pl.BlockSpec((_K, _D), lambda: (0, 0)),
            pl.BlockSpec(memory_space=pltpu.SMEM),
        ],
        out_specs=pl.BlockSpec((1, 1), lambda: (0, 0)),
        out_shape=jax.ShapeDtypeStruct((1, 1), jnp.float32),
    )(base, d3, mask3, fea, cent, scale)


def kernel(node_fea, clu_label, center_fea, mask_nodes, mask_weight,
           sort_idx_rst):
    node_fea = node_fea.astype(jnp.float32)
    center_fea = center_fea.astype(jnp.float32)
    edge_idx = sort_idx_rst[:, _S - _NE:].astype(jnp.int32)
    gidx = jnp.pad(edge_idx, ((0, 0), (0, _CP - _NE))).reshape(_B)
    m = mask_nodes.shape[0]
    allidx = jnp.concatenate([
        gidx,
        mask_nodes.astype(jnp.int32),
        jnp.full((_MP - m,), _MBUF - 8, jnp.int32),
        clu_label.astype(jnp.int32),
        jnp.zeros((_NP - _N,), jnp.int32),
    ])
    fea_pad, mask = _sc_gather_mask(node_fea, allidx)
    scale = (jnp.asarray(mask_weight, jnp.float32) + 1.0).reshape(1, 1)
    base, d1 = _tc_a(node_fea, allidx, center_fea)
    out = _tc_b(base, d1, mask, fea_pad, center_fea, scale)
    return out.reshape(1)
